# Initial kernel scaffold; baseline (speedup 1.0000x reference)
import jax
import jax.numpy as jnp
from jax.experimental import pallas as pl


def kernel(batched_context, batched_targets, table):
    raise NotImplementedError("write your pallas kernel here")


# speedup vs baseline: 1.3704x; 1.3704x over previous
import functools

import jax
import jax.numpy as jnp
from jax import lax
from jax.experimental import pallas as pl
from jax.experimental.pallas import tpu as pltpu
from jax.experimental.pallas import tpu_sc as plsc

VOCAB_SIZE = 1000
LSE_PAD = 1024

_NC = 2
_NS = 16
_L = 16
_NW = _NC * _NS


def _lse_body(table_ref, out_ref):
    x = table_ref[...]
    m = jnp.max(x, axis=1, keepdims=True)
    s = jnp.sum(jnp.exp(x - m), axis=1, keepdims=True)
    out_ref[...] = m + jnp.log(s)


def _row_logsumexp(table):
    return pl.pallas_call(
        _lse_body,
        out_shape=jax.ShapeDtypeStruct((table.shape[0], 1), jnp.float32),
    )(table)


def _sc_gather_and_loss(ctx_flat, tgt_flat, table, lse_pad):
    n_tok = ctx_flat.shape[0]
    per_w = n_tok // _NW
    chunk = 64
    n_chunk = per_w // chunk

    mesh = plsc.VectorSubcoreMesh(core_axis_name="c", subcore_axis_name="s")

    @functools.partial(
        pl.kernel,
        out_type=[
            jax.ShapeDtypeStruct((n_tok, VOCAB_SIZE), jnp.float32),
            jax.ShapeDtypeStruct((_NW, _L), jnp.float32),
        ],
        mesh=mesh,
        compiler_params=pltpu.CompilerParams(
            needs_layout_passes=False, use_tc_tiling_on_sc=False),
        scratch_types=[
            pltpu.VMEM((chunk,), jnp.int32),
            pltpu.VMEM((chunk,), jnp.int32),
            pltpu.VMEM((LSE_PAD,), jnp.float32),
            pltpu.VMEM((chunk, VOCAB_SIZE), jnp.float32),
            pltpu.VMEM((_L,), jnp.float32),
            pltpu.SemaphoreType.DMA,
        ],
    )
    def k(ctx_hbm, tgt_hbm, table_hbm, lse_hbm, out_hbm, part_hbm,
          idx_v, tgt_v, lse_v, rows_v, part_v, sem):
        wid = lax.axis_index("s") * _NC + lax.axis_index("c")
        base = wid * per_w
        pltpu.sync_copy(lse_hbm, lse_v)

        def chunk_body(g, acc):
            start = base + g * chunk
            pltpu.sync_copy(ctx_hbm.at[pl.ds(start, chunk)], idx_v)
            pltpu.sync_copy(tgt_hbm.at[pl.ds(start, chunk)], tgt_v)
            pltpu.async_copy(table_hbm.at[idx_v], rows_v, sem).wait()
            pltpu.sync_copy(rows_v, out_hbm.at[pl.ds(start, chunk)])
            for kk in range(chunk // _L):
                cvals = idx_v[pl.ds(kk * _L, _L)]
                tvals = tgt_v[pl.ds(kk * _L, _L)]
                rowids = lax.iota(jnp.int32, _L) + (kk * _L)
                lse_g = plsc.load_gather(lse_v, [cvals])
                picked = plsc.load_gather(rows_v, [rowids, tvals])
                acc = acc + (lse_g - picked)
            return acc

        acc = lax.fori_loop(0, n_chunk, chunk_body,
                            jnp.zeros((_L,), jnp.float32))
        part_v[...] = acc
        pltpu.sync_copy(part_v, part_hbm.at[wid])

    return k(ctx_flat, tgt_flat, table, lse_pad)


def kernel(batched_context, batched_targets, table):
    b, t = batched_context.shape
    ctx_flat = batched_context.reshape(-1).astype(jnp.int32)
    tgt_flat = batched_targets.reshape(-1).astype(jnp.int32)

    lse = _row_logsumexp(table)[:, 0]
    lse_pad = jnp.pad(lse, (0, LSE_PAD - VOCAB_SIZE))

    out, part = _sc_gather_and_loss(ctx_flat, tgt_flat, table, lse_pad)
    logits = out.reshape(b, t, VOCAB_SIZE)
    loss = jnp.sum(part) / (b * t)
    return (logits, loss)


# speedup vs baseline: 1.4344x; 1.0467x over previous
import functools

import jax
import jax.numpy as jnp
from jax import lax
from jax.experimental import pallas as pl
from jax.experimental.pallas import tpu as pltpu
from jax.experimental.pallas import tpu_sc as plsc

VOCAB_SIZE = 1000
LSE_PAD = 1024

_NC = 2
_NS = 16
_L = 16
_NW = _NC * _NS


def _lse_body(table_ref, out_ref):
    x = table_ref[...]
    m = jnp.max(x, axis=1, keepdims=True)
    s = jnp.sum(jnp.exp(x - m), axis=1, keepdims=True)
    out_ref[...] = m + jnp.log(s)


def _row_logsumexp(table):
    return pl.pallas_call(
        _lse_body,
        out_shape=jax.ShapeDtypeStruct((table.shape[0], 1), jnp.float32),
    )(table)


def _sc_gather_and_loss(ctx_flat, tgt_flat, table, lse_pad):
    n_tok = ctx_flat.shape[0]
    per_w = n_tok // _NW
    chunk = _L
    n_chunk = per_w // chunk
    nbuf = 4
    n_outer = n_chunk // nbuf

    mesh = plsc.VectorSubcoreMesh(core_axis_name="c", subcore_axis_name="s")

    @functools.partial(
        pl.kernel,
        out_type=[
            jax.ShapeDtypeStruct((n_tok, VOCAB_SIZE), jnp.float32),
            jax.ShapeDtypeStruct((_NW, _L), jnp.float32),
        ],
        mesh=mesh,
        compiler_params=pltpu.CompilerParams(
            needs_layout_passes=False, use_tc_tiling_on_sc=False),
        scratch_types=[
            pltpu.VMEM((per_w,), jnp.int32),
            pltpu.VMEM((per_w,), jnp.int32),
            pltpu.VMEM((VOCAB_SIZE,), jnp.float32),
            [pltpu.VMEM((chunk, VOCAB_SIZE), jnp.float32)] * nbuf,
            pltpu.VMEM((_L,), jnp.float32),
            [pltpu.SemaphoreType.DMA] * nbuf,
            [pltpu.SemaphoreType.DMA] * nbuf,
        ],
    )
    def k(ctx_hbm, tgt_hbm, table_hbm, lse_hbm, out_hbm, part_hbm,
          idx_all, tgt_all, lse_v, rows, part_v, gsem, ssem):
        wid = lax.axis_index("s") * _NC + lax.axis_index("c")
        base = wid * per_w
        pltpu.sync_copy(ctx_hbm.at[pl.ds(base, per_w)], idx_all)
        pltpu.sync_copy(tgt_hbm.at[pl.ds(base, per_w)], tgt_all)
        pltpu.sync_copy(lse_hbm, lse_v)

        def gather(g, j):
            pltpu.async_copy(
                table_hbm.at[idx_all.at[pl.ds(g * chunk, chunk)]],
                rows[j], gsem[j])

        gather(0, 0)
        gather(1, 1)

        def outer_body(outer, acc):
            for j in range(nbuf):
                g = outer * nbuf + j
                jn = (j + 2) % nbuf

                @pl.when(g + 2 < n_chunk)
                def _():
                    @pl.when(g >= 2)
                    def _():
                        pltpu.make_async_copy(
                            rows[jn],
                            out_hbm.at[pl.ds(0, chunk)], ssem[jn]).wait()
                    gather(g + 2, jn)

                pltpu.make_async_copy(
                    table_hbm.at[idx_all.at[pl.ds(0, chunk)]],
                    rows[j], gsem[j]).wait()

                cvals = idx_all[pl.ds(g * chunk, _L)]
                tvals = tgt_all[pl.ds(g * chunk, _L)]
                rowids = lax.iota(jnp.int32, _L)
                lse_g = plsc.load_gather(lse_v, [cvals])
                picked = plsc.load_gather(rows[j], [rowids, tvals])
                acc = acc + (lse_g - picked)

                pltpu.async_copy(
                    rows[j], out_hbm.at[pl.ds(base + g * chunk, chunk)],
                    ssem[j])
            return acc

        acc = lax.fori_loop(0, n_outer, outer_body,
                            jnp.zeros((_L,), jnp.float32))

        for g in (n_chunk - 2, n_chunk - 1):
            j = g % nbuf
            pltpu.make_async_copy(
                rows[j], out_hbm.at[pl.ds(0, chunk)], ssem[j]).wait()

        part_v[...] = acc
        pltpu.sync_copy(part_v, part_hbm.at[wid])

    return k(ctx_flat, tgt_flat, table, lse_pad)


def kernel(batched_context, batched_targets, table):
    b, t = batched_context.shape
    ctx_flat = batched_context.reshape(-1).astype(jnp.int32)
    tgt_flat = batched_targets.reshape(-1).astype(jnp.int32)

    lse = _row_logsumexp(table)[:, 0]

    out, part = _sc_gather_and_loss(ctx_flat, tgt_flat, table, lse)
    logits = out.reshape(b, t, VOCAB_SIZE)
    loss = jnp.sum(part) / (b * t)
    return (logits, loss)
